# baseline (speedup 1.0000x reference)
import functools

import jax
import jax.numpy as jnp
from jax import lax
from jax.experimental import pallas as pl
from jax.experimental.pallas import tpu as pltpu
from jax.experimental.pallas import tpu_sc as plsc

NC, NS, L = 2, 16, 16
NW = NC * NS

MC = 3456
G = 8


def _round_mult(v):
    y = v * jnp.float32(65537.0)
    return y - (y - v)

def _sc_body(B, N, MSC, WPB, MS,
             px_hbm, py_hbm, pz_hbm, gx_hbm, gy_hbm, gz_hbm,
             row_hbm, col_hbm,
             px_v, py_v, pz_v, gx_v, gy_v, gz_v, rowvec_v, col_v):
    nchunks = MS // L
    group_sizes = [G] * (nchunks // G)
    if nchunks % G:
        group_sizes.append(nchunks % G)
    wid = lax.axis_index("s") * NC + lax.axis_index("c")
    b = wid // WPB
    ms = (wid % WPB) * MS

    pltpu.sync_copy(px_hbm.at[pl.ds(b * N, N)], px_v)
    pltpu.sync_copy(py_hbm.at[pl.ds(b * N, N)], py_v)
    pltpu.sync_copy(pz_hbm.at[pl.ds(b * N, N)], pz_v)
    goff = b * MSC + ms
    pltpu.sync_copy(gx_hbm.at[pl.ds(goff, MS)], gx_v)
    pltpu.sync_copy(gy_hbm.at[pl.ds(goff, MS)], gy_v)
    pltpu.sync_copy(gz_hbm.at[pl.ds(goff, MS)], gz_v)

    gbase = 0
    for grp, GS in enumerate(group_sizes):
        base = gbase
        gbase += GS * L
        gxr = [gx_v[pl.ds(base + c * L, L)] for c in range(GS)]
        gyr = [gy_v[pl.ds(base + c * L, L)] for c in range(GS)]
        gzr = [gz_v[pl.ds(base + c * L, L)] for c in range(GS)]
        sq2 = [gxr[c] * gxr[c] + gyr[c] * gyr[c] + gzr[c] * gzr[c]
               for c in range(GS)]
        gx = [_round_mult(gxr[c]) for c in range(GS)]
        gy = [_round_mult(gyr[c]) for c in range(GS)]
        gz = [_round_mult(gzr[c]) for c in range(GS)]
        init = tuple(jnp.full((L,), jnp.inf, jnp.float32) for _ in range(GS))

        def nb_body(nb, cmins, grp=grp, GS=GS, gx=gx, gy=gy, gz=gz, sq2=sq2):
            nb16 = nb * L
            pxv = px_v[pl.ds(nb16, L)]
            pyv = py_v[pl.ds(nb16, L)]
            pzv = pz_v[pl.ds(nb16, L)]
            sq1v = pxv * pxv + pyv * pyv + pzv * pzv
            pxb = _round_mult(pxv)
            pyb = _round_mult(pyv)
            pzb = _round_mult(pzv)
            out = list(cmins)
            for lane in range(L):
                sq1 = sq1v[lane]
                bx = -2.0 * pxb[lane]
                by = -2.0 * pyb[lane]
                bz = -2.0 * pzb[lane]
                rmin = None
                for c in range(GS):
                    t = sq2[c] + gx[c] * bx + gy[c] * by + gz[c] * bz
                    out[c] = jnp.minimum(out[c], t + sq1)
                    rmin = t if rmin is None else jnp.minimum(rmin, t)
                rmin = rmin + sq1
                n = nb16 + lane
                if grp == 0:
                    rowvec_v[pl.ds(n * L, L)] = rmin
                else:
                    rowvec_v[pl.ds(n * L, L)] = jnp.minimum(
                        rowvec_v[pl.ds(n * L, L)], rmin)
            return tuple(out)

        cmins = lax.fori_loop(0, N // L, nb_body, init)
        for c in range(GS):
            col_v[pl.ds(base + c * L, L)] = cmins[c]

    pltpu.sync_copy(rowvec_v, row_hbm.at[pl.ds(wid * N * L, N * L)])
    pltpu.sync_copy(col_v, col_hbm.at[pl.ds(b * MSC + ms, MS)])


def _sc_call(predT, gtT_sc):
    B, _, N = predT.shape
    MSC = gtT_sc.shape[2]
    WPB = NW // B
    MS = MSC // WPB
    assert MS * WPB == MSC and MS % L == 0, (MSC, MS)
    mesh = plsc.VectorSubcoreMesh(core_axis_name="c", subcore_axis_name="s")
    f = pl.kernel(
        functools.partial(_sc_body, B, N, MSC, WPB, MS),
        out_type=[
            jax.ShapeDtypeStruct((NW * N * L,), jnp.float32),
            jax.ShapeDtypeStruct((B * MSC,), jnp.float32),
        ],
        mesh=mesh,
        scratch_types=[
            pltpu.VMEM((N,), jnp.float32),
            pltpu.VMEM((N,), jnp.float32),
            pltpu.VMEM((N,), jnp.float32),
            pltpu.VMEM((MS,), jnp.float32),
            pltpu.VMEM((MS,), jnp.float32),
            pltpu.VMEM((MS,), jnp.float32),
            pltpu.VMEM((N * L,), jnp.float32),
            pltpu.VMEM((MS,), jnp.float32),
        ],
    )
    return f(
        predT[:, 0, :].reshape(-1), predT[:, 1, :].reshape(-1),
        predT[:, 2, :].reshape(-1), gtT_sc[:, 0, :].reshape(-1),
        gtT_sc[:, 1, :].reshape(-1), gtT_sc[:, 2, :].reshape(-1),
    )


def _tc_body(p_ref, g_ref, d1_ref, d2_ref):
    i = pl.program_id(1)
    p = p_ref[0]
    g = g_ref[0]
    sq1 = jnp.sum(p * p, axis=0)
    sq2 = jnp.sum(g * g, axis=0)
    pm2 = -2.0 * p
    inner2 = jax.lax.dot_general(
        pm2, g, (((0,), (0,)), ((), ())), preferred_element_type=jnp.float32
    )
    t = inner2 + sq2[None, :]
    u = t + sq1[:, None]
    d1_ref[0, 0, :] = jnp.min(t, axis=1) + sq1
    colmin = jnp.min(u, axis=0)

    @pl.when(i == 0)
    def _():
        d2_ref[0, 0, :] = colmin

    @pl.when(i > 0)
    def _():
        d2_ref[0, 0, :] = jnp.minimum(d2_ref[0, 0, :], colmin)


def _tc_call(predT, gtT_tc):
    B, _, N = predT.shape
    Mtc = gtT_tc.shape[2]
    TN = 1024
    return pl.pallas_call(
        _tc_body,
        grid=(B, N // TN),
        in_specs=[
            pl.BlockSpec((1, 3, TN), lambda b, i: (b, 0, i)),
            pl.BlockSpec((1, 3, Mtc), lambda b, i: (b, 0, 0)),
        ],
        out_specs=[
            pl.BlockSpec((1, 1, TN), lambda b, i: (b, 0, i)),
            pl.BlockSpec((1, 1, Mtc), lambda b, i: (b, 0, 0)),
        ],
        out_shape=[
            jax.ShapeDtypeStruct((B, 1, N), jnp.float32),
            jax.ShapeDtypeStruct((B, 1, Mtc), jnp.float32),
        ],
    )(predT, gtT_tc)


@jax.jit
def kernel(pred, gt):
    B, N, _ = pred.shape
    M = gt.shape[1]
    predT = jnp.swapaxes(pred, 1, 2)
    gtT = jnp.swapaxes(gt, 1, 2)

    parts1 = []
    parts2 = []
    if MC < M:
        row_sc, col_sc = _sc_call(predT, gtT[:, :, MC:])
        WPB = NW // B
        r = functools.reduce(
            jnp.minimum, [row_sc[l::L] for l in range(L)])
        parts1.append(jnp.min(r.reshape(B, WPB, N), axis=1))
        parts2.append(col_sc.reshape(B, M - MC))
    if MC > 0:
        d1_tc, d2_tc = _tc_call(predT, gtT[:, :, :MC])
        parts1.append(d1_tc[:, 0, :])
        parts2.insert(0, d2_tc[:, 0, :])

    dist1 = functools.reduce(jnp.minimum, parts1)
    dist2 = jnp.concatenate(parts2, axis=1) if len(parts2) > 1 else parts2[0]
    return jnp.mean(dist1) + jnp.mean(dist2)


# speedup vs baseline: 4.6972x; 4.6972x over previous
import functools

import jax
import jax.numpy as jnp
from jax import lax
from jax.experimental import pallas as pl
from jax.experimental.pallas import tpu as pltpu
from jax.experimental.pallas import tpu_sc as plsc

NC, NS, L = 2, 16, 16
NW = NC * NS

MC = 3456
G = 8


def _round_mult(v):
    y = v * jnp.float32(65537.0)
    return y - (y - v)

def _sc_body(B, N, MSC, WPB, MS,
             px_hbm, py_hbm, pz_hbm, gx_hbm, gy_hbm, gz_hbm,
             row_hbm, col_hbm,
             px_v, py_v, pz_v, gx_v, gy_v, gz_v, rowvec_v, col_v):
    nchunks = MS // L
    group_sizes = [G] * (nchunks // G)
    if nchunks % G:
        group_sizes.append(nchunks % G)
    wid = lax.axis_index("s") * NC + lax.axis_index("c")
    b = wid // WPB
    ms = (wid % WPB) * MS

    pltpu.sync_copy(px_hbm.at[pl.ds(b * N, N)], px_v)
    pltpu.sync_copy(py_hbm.at[pl.ds(b * N, N)], py_v)
    pltpu.sync_copy(pz_hbm.at[pl.ds(b * N, N)], pz_v)
    goff = b * MSC + ms
    pltpu.sync_copy(gx_hbm.at[pl.ds(goff, MS)], gx_v)
    pltpu.sync_copy(gy_hbm.at[pl.ds(goff, MS)], gy_v)
    pltpu.sync_copy(gz_hbm.at[pl.ds(goff, MS)], gz_v)

    gbase = 0
    for grp, GS in enumerate(group_sizes):
        base = gbase
        gbase += GS * L
        gxr = [gx_v[pl.ds(base + c * L, L)] for c in range(GS)]
        gyr = [gy_v[pl.ds(base + c * L, L)] for c in range(GS)]
        gzr = [gz_v[pl.ds(base + c * L, L)] for c in range(GS)]
        sq2 = [gxr[c] * gxr[c] + gyr[c] * gyr[c] + gzr[c] * gzr[c]
               for c in range(GS)]
        gx = [_round_mult(gxr[c]) for c in range(GS)]
        gy = [_round_mult(gyr[c]) for c in range(GS)]
        gz = [_round_mult(gzr[c]) for c in range(GS)]
        init = tuple(jnp.full((L,), jnp.inf, jnp.float32) for _ in range(GS))

        def nb_body(nb, cmins, grp=grp, GS=GS, gx=gx, gy=gy, gz=gz, sq2=sq2):
            nb16 = nb * L
            pxv = px_v[pl.ds(nb16, L)]
            pyv = py_v[pl.ds(nb16, L)]
            pzv = pz_v[pl.ds(nb16, L)]
            sq1v = pxv * pxv + pyv * pyv + pzv * pzv
            pxb = _round_mult(pxv)
            pyb = _round_mult(pyv)
            pzb = _round_mult(pzv)
            out = list(cmins)
            for lane in range(L):
                sq1 = sq1v[lane]
                bx = -2.0 * pxb[lane]
                by = -2.0 * pyb[lane]
                bz = -2.0 * pzb[lane]
                rmin = None
                for c in range(GS):
                    t = sq2[c] + gx[c] * bx + gy[c] * by + gz[c] * bz
                    out[c] = jnp.minimum(out[c], t + sq1)
                    rmin = t if rmin is None else jnp.minimum(rmin, t)
                rmin = rmin + sq1
                n = nb16 + lane
                if grp == 0:
                    rowvec_v[pl.ds(n * L, L)] = rmin
                else:
                    rowvec_v[pl.ds(n * L, L)] = jnp.minimum(
                        rowvec_v[pl.ds(n * L, L)], rmin)
            return tuple(out)

        cmins = lax.fori_loop(0, N // L, nb_body, init)
        for c in range(GS):
            col_v[pl.ds(base + c * L, L)] = cmins[c]

    pltpu.sync_copy(rowvec_v, row_hbm.at[pl.ds(wid * N * L, N * L)])
    pltpu.sync_copy(col_v, col_hbm.at[pl.ds(b * MSC + ms, MS)])


def _sc_call(predT, gtT_sc):
    B, _, N = predT.shape
    MSC = gtT_sc.shape[2]
    WPB = NW // B
    MS = MSC // WPB
    assert MS * WPB == MSC and MS % L == 0, (MSC, MS)
    mesh = plsc.VectorSubcoreMesh(core_axis_name="c", subcore_axis_name="s")
    f = pl.kernel(
        functools.partial(_sc_body, B, N, MSC, WPB, MS),
        out_type=[
            jax.ShapeDtypeStruct((NW * N * L,), jnp.float32),
            jax.ShapeDtypeStruct((B * MSC,), jnp.float32),
        ],
        mesh=mesh,
        scratch_types=[
            pltpu.VMEM((N,), jnp.float32),
            pltpu.VMEM((N,), jnp.float32),
            pltpu.VMEM((N,), jnp.float32),
            pltpu.VMEM((MS,), jnp.float32),
            pltpu.VMEM((MS,), jnp.float32),
            pltpu.VMEM((MS,), jnp.float32),
            pltpu.VMEM((N * L,), jnp.float32),
            pltpu.VMEM((MS,), jnp.float32),
        ],
    )
    return f(
        predT[:, 0, :].reshape(-1), predT[:, 1, :].reshape(-1),
        predT[:, 2, :].reshape(-1), gtT_sc[:, 0, :].reshape(-1),
        gtT_sc[:, 1, :].reshape(-1), gtT_sc[:, 2, :].reshape(-1),
    )


_CH = 8192


def _red_body(y_ref, o_ref):
    x = y_ref[0] if y_ref.shape[0] == 1 else y_ref[...]
    y = jnp.min(x, axis=0)
    y2 = y.reshape(_CH // 128, 128)
    for k in (8, 4, 2, 1):
        yr = jnp.concatenate([y2[:, k:], y2[:, :k]], axis=1)
        y2 = jnp.minimum(y2, yr)
    csel = jax.lax.broadcasted_iota(jnp.int32, (128, 8), 0)
    rsel = jax.lax.broadcasted_iota(jnp.int32, (128, 8), 1)
    sel = (csel == rsel * 16).astype(jnp.float32)
    z = jax.lax.dot_general(
        y2, sel, (((1,), (0,)), ((), ())), preferred_element_type=jnp.float32
    )
    o_ref[0, 0] = z


def _lane_reduce_call(row_sc, B, N):
    WPB = NW // B
    NI = (N * L) // _CH
    row2 = row_sc.reshape(NW, N * L)
    out = pl.pallas_call(
        _red_body,
        grid=(B, NI),
        in_specs=[pl.BlockSpec((WPB, _CH), lambda b, i: (b, i))],
        out_specs=pl.BlockSpec((1, 1, _CH // 128, 8), lambda b, i: (b, i, 0, 0)),
        out_shape=jax.ShapeDtypeStruct((B, NI, _CH // 128, 8), jnp.float32),
    )(row2)
    return out.reshape(B, N)


def _tc_body(p_ref, g_ref, d1_ref, d2_ref):
    i = pl.program_id(1)
    p = p_ref[0]
    g = g_ref[0]
    sq1 = jnp.sum(p * p, axis=0)
    sq2 = jnp.sum(g * g, axis=0)
    pm2 = -2.0 * p
    inner2 = jax.lax.dot_general(
        pm2, g, (((0,), (0,)), ((), ())), preferred_element_type=jnp.float32
    )
    t = inner2 + sq2[None, :]
    u = t + sq1[:, None]
    d1_ref[0, 0, :] = jnp.min(t, axis=1) + sq1
    colmin = jnp.min(u, axis=0)

    @pl.when(i == 0)
    def _():
        d2_ref[0, 0, :] = colmin

    @pl.when(i > 0)
    def _():
        d2_ref[0, 0, :] = jnp.minimum(d2_ref[0, 0, :], colmin)


def _tc_call(predT, gtT_tc):
    B, _, N = predT.shape
    Mtc = gtT_tc.shape[2]
    TN = 1024
    return pl.pallas_call(
        _tc_body,
        grid=(B, N // TN),
        in_specs=[
            pl.BlockSpec((1, 3, TN), lambda b, i: (b, 0, i)),
            pl.BlockSpec((1, 3, Mtc), lambda b, i: (b, 0, 0)),
        ],
        out_specs=[
            pl.BlockSpec((1, 1, TN), lambda b, i: (b, 0, i)),
            pl.BlockSpec((1, 1, Mtc), lambda b, i: (b, 0, 0)),
        ],
        out_shape=[
            jax.ShapeDtypeStruct((B, 1, N), jnp.float32),
            jax.ShapeDtypeStruct((B, 1, Mtc), jnp.float32),
        ],
    )(predT, gtT_tc)


@jax.jit
def kernel(pred, gt):
    B, N, _ = pred.shape
    M = gt.shape[1]
    predT = jnp.swapaxes(pred, 1, 2)
    gtT = jnp.swapaxes(gt, 1, 2)

    parts1 = []
    parts2 = []
    if MC < M:
        row_sc, col_sc = _sc_call(predT, gtT[:, :, MC:])
        parts1.append(_lane_reduce_call(row_sc, B, N))
        parts2.append(col_sc.reshape(B, M - MC))
    if MC > 0:
        d1_tc, d2_tc = _tc_call(predT, gtT[:, :, :MC])
        parts1.append(d1_tc[:, 0, :])
        parts2.insert(0, d2_tc[:, 0, :])

    dist1 = functools.reduce(jnp.minimum, parts1)
    dist2 = jnp.concatenate(parts2, axis=1) if len(parts2) > 1 else parts2[0]
    return jnp.mean(dist1) + jnp.mean(dist2)


# speedup vs baseline: 6.0148x; 1.2805x over previous
import functools

import jax
import jax.numpy as jnp
from jax import lax
from jax.experimental import pallas as pl
from jax.experimental.pallas import tpu as pltpu
from jax.experimental.pallas import tpu_sc as plsc

NC, NS, L = 2, 16, 16
NW = NC * NS

MC = 3456
G = 8


def _round_mult(v):
    y = v * jnp.float32(65537.0)
    return y - (y - v)

def _sc_body(B, N, MSC, WPB, MS,
             px_hbm, py_hbm, pz_hbm, gx_hbm, gy_hbm, gz_hbm,
             row_hbm, col_hbm,
             px_v, py_v, pz_v, gx_v, gy_v, gz_v, rowvec_v, col_v):
    nchunks = MS // L
    group_sizes = [G] * (nchunks // G)
    if nchunks % G:
        group_sizes.append(nchunks % G)
    wid = lax.axis_index("s") * NC + lax.axis_index("c")
    b = wid // WPB
    ms = (wid % WPB) * MS

    pltpu.sync_copy(px_hbm.at[pl.ds(b * N, N)], px_v)
    pltpu.sync_copy(py_hbm.at[pl.ds(b * N, N)], py_v)
    pltpu.sync_copy(pz_hbm.at[pl.ds(b * N, N)], pz_v)
    goff = b * MSC + ms
    pltpu.sync_copy(gx_hbm.at[pl.ds(goff, MS)], gx_v)
    pltpu.sync_copy(gy_hbm.at[pl.ds(goff, MS)], gy_v)
    pltpu.sync_copy(gz_hbm.at[pl.ds(goff, MS)], gz_v)

    gbase = 0
    for grp, GS in enumerate(group_sizes):
        base = gbase
        gbase += GS * L
        gxr = [gx_v[pl.ds(base + c * L, L)] for c in range(GS)]
        gyr = [gy_v[pl.ds(base + c * L, L)] for c in range(GS)]
        gzr = [gz_v[pl.ds(base + c * L, L)] for c in range(GS)]
        sq2 = [gxr[c] * gxr[c] + gyr[c] * gyr[c] + gzr[c] * gzr[c]
               for c in range(GS)]
        gx = [_round_mult(gxr[c]) for c in range(GS)]
        gy = [_round_mult(gyr[c]) for c in range(GS)]
        gz = [_round_mult(gzr[c]) for c in range(GS)]
        init = tuple(jnp.full((L,), jnp.inf, jnp.float32) for _ in range(GS))

        def nb_body(nb, cmins, grp=grp, GS=GS, gx=gx, gy=gy, gz=gz, sq2=sq2):
            nb16 = nb * L
            pxv = px_v[pl.ds(nb16, L)]
            pyv = py_v[pl.ds(nb16, L)]
            pzv = pz_v[pl.ds(nb16, L)]
            sq1v = pxv * pxv + pyv * pyv + pzv * pzv
            pxb = _round_mult(pxv)
            pyb = _round_mult(pyv)
            pzb = _round_mult(pzv)
            out = list(cmins)
            for lane in range(L):
                sq1 = sq1v[lane]
                bx = -2.0 * pxb[lane]
                by = -2.0 * pyb[lane]
                bz = -2.0 * pzb[lane]
                rmin = None
                for c in range(GS):
                    t = sq2[c] + gx[c] * bx + gy[c] * by + gz[c] * bz
                    out[c] = jnp.minimum(out[c], t + sq1)
                    rmin = t if rmin is None else jnp.minimum(rmin, t)
                rmin = rmin + sq1
                n = nb16 + lane
                if grp == 0:
                    rowvec_v[0, pl.ds(n * L, L)] = rmin
                else:
                    rowvec_v[0, pl.ds(n * L, L)] = jnp.minimum(
                        rowvec_v[0, pl.ds(n * L, L)], rmin)
            return tuple(out)

        cmins = lax.fori_loop(0, N // L, nb_body, init)
        for c in range(GS):
            col_v[pl.ds(base + c * L, L)] = cmins[c]

    pltpu.sync_copy(rowvec_v, row_hbm.at[pl.ds(wid, 1), :])
    pltpu.sync_copy(col_v, col_hbm.at[pl.ds(b * MSC + ms, MS)])


def _sc_call(predT, gtT_sc):
    B, _, N = predT.shape
    MSC = gtT_sc.shape[2]
    WPB = NW // B
    MS = MSC // WPB
    assert MS * WPB == MSC and MS % L == 0, (MSC, MS)
    mesh = plsc.VectorSubcoreMesh(core_axis_name="c", subcore_axis_name="s")
    f = pl.kernel(
        functools.partial(_sc_body, B, N, MSC, WPB, MS),
        out_type=[
            jax.ShapeDtypeStruct((NW, N * L), jnp.float32),
            jax.ShapeDtypeStruct((B * MSC,), jnp.float32),
        ],
        mesh=mesh,
        scratch_types=[
            pltpu.VMEM((N,), jnp.float32),
            pltpu.VMEM((N,), jnp.float32),
            pltpu.VMEM((N,), jnp.float32),
            pltpu.VMEM((MS,), jnp.float32),
            pltpu.VMEM((MS,), jnp.float32),
            pltpu.VMEM((MS,), jnp.float32),
            pltpu.VMEM((1, N * L), jnp.float32),
            pltpu.VMEM((MS,), jnp.float32),
        ],
    )
    return f(
        predT[:, 0, :].reshape(-1), predT[:, 1, :].reshape(-1),
        predT[:, 2, :].reshape(-1), gtT_sc[:, 0, :].reshape(-1),
        gtT_sc[:, 1, :].reshape(-1), gtT_sc[:, 2, :].reshape(-1),
    )


def _red_body(y_ref, o_ref):
    x = y_ref[...]
    y = jnp.min(x, axis=0)
    y2 = y.reshape(y.shape[0] // 128, 128)
    for k in (8, 4, 2, 1):
        yr = jnp.concatenate([y2[:, k:], y2[:, :k]], axis=1)
        y2 = jnp.minimum(y2, yr)
    csel = jax.lax.broadcasted_iota(jnp.int32, (128, 8), 0)
    rsel = jax.lax.broadcasted_iota(jnp.int32, (128, 8), 1)
    sel = (csel == rsel * 16).astype(jnp.float32)
    z = jax.lax.dot_general(
        y2, sel, (((1,), (0,)), ((), ())), preferred_element_type=jnp.float32
    )
    o_ref[0] = z


def _lane_reduce_call(row_sc, B, N):
    WPB = NW // B
    out = pl.pallas_call(
        _red_body,
        grid=(B,),
        in_specs=[pl.BlockSpec((WPB, N * L), lambda b: (b, 0))],
        out_specs=pl.BlockSpec((1, (N * L) // 128, 8), lambda b: (b, 0, 0)),
        out_shape=jax.ShapeDtypeStruct((B, (N * L) // 128, 8), jnp.float32),
    )(row_sc)
    return out.reshape(B, N)


def _tc_body(p_ref, g_ref, d1_ref, d2_ref):
    i = pl.program_id(1)
    p = p_ref[0]
    g = g_ref[0]
    sq1 = jnp.sum(p * p, axis=0)
    sq2 = jnp.sum(g * g, axis=0)
    pm2 = -2.0 * p
    inner2 = jax.lax.dot_general(
        pm2, g, (((0,), (0,)), ((), ())), preferred_element_type=jnp.float32
    )
    t = inner2 + sq2[None, :]
    u = t + sq1[:, None]
    d1_ref[0, 0, :] = jnp.min(t, axis=1) + sq1
    colmin = jnp.min(u, axis=0)

    @pl.when(i == 0)
    def _():
        d2_ref[0, 0, :] = colmin

    @pl.when(i > 0)
    def _():
        d2_ref[0, 0, :] = jnp.minimum(d2_ref[0, 0, :], colmin)


def _tc_call(predT, gtT_tc):
    B, _, N = predT.shape
    Mtc = gtT_tc.shape[2]
    TN = 1024
    return pl.pallas_call(
        _tc_body,
        grid=(B, N // TN),
        in_specs=[
            pl.BlockSpec((1, 3, TN), lambda b, i: (b, 0, i)),
            pl.BlockSpec((1, 3, Mtc), lambda b, i: (b, 0, 0)),
        ],
        out_specs=[
            pl.BlockSpec((1, 1, TN), lambda b, i: (b, 0, i)),
            pl.BlockSpec((1, 1, Mtc), lambda b, i: (b, 0, 0)),
        ],
        out_shape=[
            jax.ShapeDtypeStruct((B, 1, N), jnp.float32),
            jax.ShapeDtypeStruct((B, 1, Mtc), jnp.float32),
        ],
    )(predT, gtT_tc)


@jax.jit
def kernel(pred, gt):
    B, N, _ = pred.shape
    M = gt.shape[1]
    predT = jnp.swapaxes(pred, 1, 2)
    gtT = jnp.swapaxes(gt, 1, 2)

    parts1 = []
    parts2 = []
    if MC < M:
        row_sc, col_sc = _sc_call(predT, gtT[:, :, MC:])
        parts1.append(_lane_reduce_call(row_sc, B, N))
        parts2.append(col_sc.reshape(B, M - MC))
    if MC > 0:
        d1_tc, d2_tc = _tc_call(predT, gtT[:, :, :MC])
        parts1.append(d1_tc[:, 0, :])
        parts2.insert(0, d2_tc[:, 0, :])

    dist1 = functools.reduce(jnp.minimum, parts1)
    dist2 = jnp.concatenate(parts2, axis=1) if len(parts2) > 1 else parts2[0]
    return jnp.mean(dist1) + jnp.mean(dist2)
